# baseline (speedup 1.0000x reference)
import functools

import jax
import jax.numpy as jnp
from jax import lax
from jax.experimental import pallas as pl
from jax.experimental.pallas import tpu as pltpu
from jax.experimental.pallas import tpu_sc as plsc

_NC = 2
_NS = 16
_NW = _NC * _NS
_L = 16
_CHUNK = 128
_TOTAL = 8 * 16384
_NPW = _TOTAL // _NW
_NCH = _NPW // _CHUNK


def _rsqrt(x):
    xi = plsc.bitcast(x, jnp.int32)
    yi = jnp.int32(0x5F3759DF) - (xi >> 1)
    y = plsc.bitcast(yi, jnp.float32)
    for _ in range(3):
        y = y * (1.5 - 0.5 * x * y * y)
    return y


def _make_kernel():
    mesh = plsc.VectorSubcoreMesh(core_axis_name="c", subcore_axis_name="s")

    @functools.partial(
        pl.kernel,
        mesh=mesh,
        compiler_params=pltpu.CompilerParams(
            needs_layout_passes=False, use_tc_tiling_on_sc=False),
        out_type=jax.ShapeDtypeStruct((_NW, _L), jnp.float32),
        scratch_types=[
            pltpu.VMEM((_NCH, _CHUNK), jnp.int32),
            pltpu.VMEM((_NPW, _L), jnp.float32),
            pltpu.VMEM((_L,), jnp.float32),
            pltpu.SemaphoreType.DMA,
        ],
    )
    def k(table_hbm, idx_hbm, out_hbm, idx_v, rows_v, acc_v, sem):
        wid = lax.axis_index("s") * _NC + lax.axis_index("c")
        pltpu.sync_copy(idx_hbm.at[wid], idx_v)
        copies = [
            pltpu.async_copy(table_hbm.at[idx_v.at[j]],
                             rows_v.at[pl.ds(j * _CHUNK, _CHUNK)], sem)
            for j in range(_NCH)
        ]
        for cp in copies:
            cp.wait()

        iota = lax.iota(jnp.int32, _L)

        def chunk_body(m, acc):
            mbase = m * _CHUNK
            for gi in range(_CHUNK // _L):
                rows = iota + (mbase + gi * _L)
                ssq = jnp.zeros((_L,), jnp.float32)
                for c in range(_L):
                    col = jnp.full((_L,), c, dtype=jnp.int32)
                    g = plsc.load_gather(rows_v, [rows, col])
                    ssq = ssq + g * g
                acc = acc + ssq * _rsqrt(ssq)
            return acc

        acc = lax.fori_loop(0, _NCH, chunk_body,
                            jnp.zeros((_L,), jnp.float32))
        acc_v[...] = acc
        pltpu.sync_copy(acc_v, out_hbm.at[wid])

    return k


_sc_kernel = _make_kernel()


def kernel(table, indices):
    idx = indices.reshape(_NW, _NCH, _CHUNK).astype(jnp.int32)
    partials = _sc_kernel(table, idx)
    return jnp.sum(partials) * (1.0 / _TOTAL)
